# baseline (speedup 1.0000x reference)
import jax
import jax.numpy as jnp
from jax.experimental import pallas as pl

_NEG = 0.2


def _pad_rows(x, m):
    r = (-x.shape[0]) % m
    return jnp.pad(x, ((0, r), (0, 0))) if r else x


def _mm_kern(x_ref, w_ref, o_ref):
    o_ref[...] = jnp.dot(x_ref[...], w_ref[...],
                         preferred_element_type=jnp.float32)


def _matmul(x, w, tm=512):
    n, k = x.shape
    m = w.shape[1]
    xp = _pad_rows(x, tm)
    wp = jnp.pad(w, ((0, 0), (0, (-m) % 128)))
    gm, gn = xp.shape[0] // tm, wp.shape[1] // 128
    out = pl.pallas_call(
        _mm_kern,
        grid=(gm, gn),
        in_specs=[pl.BlockSpec((tm, k), lambda i, j: (i, 0)),
                  pl.BlockSpec((k, 128), lambda i, j: (0, j))],
        out_specs=pl.BlockSpec((tm, 128), lambda i, j: (i, j)),
        out_shape=jax.ShapeDtypeStruct((xp.shape[0], wp.shape[1]),
                                       jnp.float32),
    )(xp, wp)
    return out[:n, :m]


def _logits_kern(a_ref, b_ref, o_ref):
    s = a_ref[...] + b_ref[...]
    o_ref[...] = jnp.where(s >= 0, s, _NEG * s)


def _edge_logits(la, lb):
    e = la.shape[0]
    ep = -(-e // 512) * 512
    a = jnp.pad(la, (0, ep - e)).reshape(-1, 512)
    b = jnp.pad(lb, (0, ep - e)).reshape(-1, 512)
    o = pl.pallas_call(
        _logits_kern,
        out_shape=jax.ShapeDtypeStruct(a.shape, jnp.float32))(a, b)
    return o.reshape(-1)[:e]


def _exp_kern(l_ref, m_ref, o_ref):
    o_ref[...] = jnp.exp(l_ref[...] - m_ref[...])


def _edge_exp(lg, mg):
    e = lg.shape[0]
    ep = -(-e // 512) * 512
    a = jnp.pad(lg, (0, ep - e)).reshape(-1, 512)
    b = jnp.pad(mg, (0, ep - e)).reshape(-1, 512)
    o = pl.pallas_call(
        _exp_kern,
        out_shape=jax.ShapeDtypeStruct(a.shape, jnp.float32))(a, b)
    return o.reshape(-1)[:e]


def _wval_kern(e_ref, s_ref, v_ref, o_ref):
    att = e_ref[...] / (s_ref[...] + 1e-12)
    o_ref[...] = att * v_ref[...]


def _weighted_vals(e, s_g, vals):
    n = e.shape[0]
    npad = -(-n // 512) * 512
    ep = jnp.pad(e, (0, npad - n)).reshape(-1, 1)
    sp = jnp.pad(s_g, (0, npad - n), constant_values=1.0).reshape(-1, 1)
    vp = _pad_rows(vals, 512)
    g = npad // 512
    o = pl.pallas_call(
        _wval_kern,
        grid=(g,),
        in_specs=[pl.BlockSpec((512, 1), lambda i: (i, 0)),
                  pl.BlockSpec((512, 1), lambda i: (i, 0)),
                  pl.BlockSpec((512, 128), lambda i: (i, 0))],
        out_specs=pl.BlockSpec((512, 128), lambda i: (i, 0)),
        out_shape=jax.ShapeDtypeStruct((npad, 128), jnp.float32),
    )(ep, sp, vp)
    return o[:n]


def _comb1_kern(a_ref, o_ref):
    o_ref[...] = jnp.maximum(a_ref[...], 0.0)


def _comb2_kern(a_ref, b_ref, o_ref):
    o_ref[...] = jnp.maximum(a_ref[...], 0.0) + jnp.maximum(b_ref[...], 0.0)


def _comb3_kern(a_ref, b_ref, c_ref, o_ref):
    o_ref[...] = (jnp.maximum(a_ref[...], 0.0)
                  + jnp.maximum(b_ref[...], 0.0)
                  + jnp.maximum(c_ref[...], 0.0))


def _combine(*arrs):
    n = arrs[0].shape[0]
    tm = 512
    ps = [_pad_rows(a, tm) for a in arrs]
    g = ps[0].shape[0] // tm
    kern = {1: _comb1_kern, 2: _comb2_kern, 3: _comb3_kern}[len(arrs)]
    o = pl.pallas_call(
        kern,
        grid=(g,),
        in_specs=[pl.BlockSpec((tm, 128), lambda i: (i, 0))] * len(arrs),
        out_specs=pl.BlockSpec((tm, 128), lambda i: (i, 0)),
        out_shape=jax.ShapeDtypeStruct(ps[0].shape, jnp.float32),
    )(*ps)
    return o[:n]


def _att_reduce(la, lb, vals_g, seg, n, seg_sorted):
    lg = _edge_logits(la, lb)
    m = jax.ops.segment_max(lg, seg, num_segments=n,
                            indices_are_sorted=seg_sorted)
    m = jnp.where(jnp.isfinite(m), m, 0.0)
    e = _edge_exp(lg, jnp.take(m, seg))
    s = jax.ops.segment_sum(e, seg, num_segments=n,
                            indices_are_sorted=seg_sorted)
    wv = _weighted_vals(e, jnp.take(s, seg), vals_g)
    return jax.ops.segment_sum(wv, seg, num_segments=n,
                               indices_are_sorted=seg_sorted)


def _split_a(a):
    d = a.shape[0] // 2
    return a[:d], a[d:]


def kernel(x_0, x_1, x_2, params, neighborhood_0_to_0, neighborhood_1_to_1,
           neighborhood_2_to_2, neighborhood_0_to_1, neighborhood_1_to_2):
    p = params
    n0, d = x_0.shape
    n1 = x_1.shape[0]
    n2 = x_2.shape[0]
    r00, c00 = neighborhood_0_to_0[0], neighborhood_0_to_0[1]
    r11, c11 = neighborhood_1_to_1[0], neighborhood_1_to_1[1]
    r22, c22 = neighborhood_2_to_2[0], neighborhood_2_to_2[1]
    r01, c01 = neighborhood_0_to_1[0], neighborhood_0_to_1[1]
    r12, c12 = neighborhood_1_to_2[0], neighborhood_1_to_2[1]

    a0t, a0b = _split_a(p["hbs0_l1_a"])
    a01t, a01b = _split_a(p["hbns01_l1_a"])
    a12t, a12b = _split_a(p["hbns12_l1_a"])
    w0 = p["hbs0_l1_w"]
    ws01, wt01 = p["hbns01_l1_ws"], p["hbns01_l1_wt"]
    ws12, wt12 = p["hbns12_l1_ws"], p["hbns12_l1_wt"]

    m0 = _matmul(x_0, jnp.concatenate(
        [w0, wt01, w0 @ a0t, w0 @ a0b, wt01 @ a01t, wt01 @ a01b], axis=1))
    msg00, t01 = m0[:, :d], m0[:, d:2 * d]
    u00, v00 = m0[:, 2 * d], m0[:, 2 * d + 1]
    t01u1, t01u2 = m0[:, 2 * d + 2], m0[:, 2 * d + 3]

    m1 = _matmul(x_1, jnp.concatenate(
        [ws01, wt12, ws01 @ a01t, ws01 @ a01b, wt12 @ a12t, wt12 @ a12b],
        axis=1))
    s01, t12 = m1[:, :d], m1[:, d:2 * d]
    s01u1, s01u2 = m1[:, 2 * d], m1[:, 2 * d + 1]
    t12u1, t12u2 = m1[:, 2 * d + 2], m1[:, 2 * d + 3]

    m2 = _matmul(x_2, jnp.concatenate(
        [ws12, ws12 @ a12t, ws12 @ a12b], axis=1))
    s12 = m2[:, :d]
    s12u1, s12u2 = m2[:, d], m2[:, d + 1]

    x0_l1 = _att_reduce(jnp.take(u00, r00), jnp.take(v00, c00),
                        jnp.take(msg00, c00, axis=0), r00, n0, True)
    x10_l1 = _att_reduce(jnp.take(s01u1, c01), jnp.take(t01u2, r01),
                         jnp.take(s01, c01, axis=0), r01, n0, True)
    x01_l1 = _att_reduce(jnp.take(t01u1, r01), jnp.take(s01u2, c01),
                         jnp.take(t01, r01, axis=0), c01, n1, False)
    x21_l1 = _att_reduce(jnp.take(s12u1, c12), jnp.take(t12u2, r12),
                         jnp.take(s12, c12, axis=0), r12, n1, True)
    x12_l1 = _att_reduce(jnp.take(t12u1, r12), jnp.take(s12u2, c12),
                         jnp.take(t12, r12, axis=0), c12, n2, False)

    h0 = _combine(x0_l1, x10_l1)
    h1 = _combine(x01_l1, x21_l1)
    h2 = _combine(x12_l1)

    b0t, b0b = _split_a(p["hbs0_l2_a"])
    b1t, b1b = _split_a(p["hbs1_l2_a"])
    b2t, b2b = _split_a(p["hbs2_l2_a"])
    b01t, b01b = _split_a(p["hbns01_l2_a"])
    b12t, b12b = _split_a(p["hbns12_l2_a"])
    q0 = p["hbs0_l2_w"]
    q1 = p["hbs1_l2_w"]
    q2 = p["hbs2_l2_w"]
    qs01, qt01 = p["hbns01_l2_ws"], p["hbns01_l2_wt"]
    qs12, qt12 = p["hbns12_l2_ws"], p["hbns12_l2_wt"]

    k0 = _matmul(h0, jnp.concatenate(
        [q0, qt01, q0 @ b0t, q0 @ b0b, qt01 @ b01t, qt01 @ b01b], axis=1))
    g00, gt01 = k0[:, :d], k0[:, d:2 * d]
    gu00, gv00 = k0[:, 2 * d], k0[:, 2 * d + 1]
    gt01u1, gt01u2 = k0[:, 2 * d + 2], k0[:, 2 * d + 3]

    k1 = _matmul(h1, jnp.concatenate(
        [qs01, q1, qt12, qs01 @ b01t, qs01 @ b01b, q1 @ b1t, q1 @ b1b,
         qt12 @ b12t, qt12 @ b12b], axis=1))
    gs01, g11, gt12 = k1[:, :d], k1[:, d:2 * d], k1[:, 2 * d:3 * d]
    gs01u1, gs01u2 = k1[:, 3 * d], k1[:, 3 * d + 1]
    gu11, gv11 = k1[:, 3 * d + 2], k1[:, 3 * d + 3]
    gt12u1, gt12u2 = k1[:, 3 * d + 4], k1[:, 3 * d + 5]

    k2 = _matmul(h2, jnp.concatenate(
        [qs12, q2, qs12 @ b12t, qs12 @ b12b, q2 @ b2t, q2 @ b2b], axis=1))
    gs12, g22 = k2[:, :d], k2[:, d:2 * d]
    gs12u1, gs12u2 = k2[:, 2 * d], k2[:, 2 * d + 1]
    gu22, gv22 = k2[:, 2 * d + 2], k2[:, 2 * d + 3]

    x0_l2 = _att_reduce(jnp.take(gu00, r00), jnp.take(gv00, c00),
                        jnp.take(g00, c00, axis=0), r00, n0, True)
    x10_l2 = _att_reduce(jnp.take(gs01u1, c01), jnp.take(gt01u2, r01),
                         jnp.take(gs01, c01, axis=0), r01, n0, True)
    x01_l2 = _att_reduce(jnp.take(gt01u1, r01), jnp.take(gs01u2, c01),
                         jnp.take(gt01, r01, axis=0), c01, n1, False)
    x1_l2 = _att_reduce(jnp.take(gu11, r11), jnp.take(gv11, c11),
                        jnp.take(g11, c11, axis=0), r11, n1, True)
    x21_l2 = _att_reduce(jnp.take(gs12u1, c12), jnp.take(gt12u2, r12),
                         jnp.take(gs12, c12, axis=0), r12, n1, True)
    x12_l2 = _att_reduce(jnp.take(gt12u1, r12), jnp.take(gs12u2, c12),
                         jnp.take(gt12, r12, axis=0), c12, n2, False)
    x2_l2 = _att_reduce(jnp.take(gu22, r22), jnp.take(gv22, c22),
                        jnp.take(g22, c22, axis=0), r22, n2, True)

    out_0 = _combine(x0_l2, x10_l2)
    out_1 = _combine(x01_l2, x1_l2, x21_l2)
    out_2 = _combine(x12_l2, x2_l2)
    return (out_0, out_1, out_2)


# speedup vs baseline: 1.0759x; 1.0759x over previous
import jax
import jax.numpy as jnp
from jax.experimental import pallas as pl

_NEG = 0.2


def _pad_rows(x, m):
    r = (-x.shape[0]) % m
    return jnp.pad(x, ((0, r), (0, 0))) if r else x


def _mm_kern(x_ref, w_ref, o_ref):
    o_ref[...] = jnp.dot(x_ref[...], w_ref[...],
                         preferred_element_type=jnp.float32)


def _matmul(x, w, tm=512):
    n, k = x.shape
    m = w.shape[1]
    xp = _pad_rows(x, tm)
    wp = jnp.pad(w, ((0, 0), (0, (-m) % 128)))
    gm, gn = xp.shape[0] // tm, wp.shape[1] // 128
    out = pl.pallas_call(
        _mm_kern,
        grid=(gm, gn),
        in_specs=[pl.BlockSpec((tm, k), lambda i, j: (i, 0)),
                  pl.BlockSpec((k, 128), lambda i, j: (0, j))],
        out_specs=pl.BlockSpec((tm, 128), lambda i, j: (i, j)),
        out_shape=jax.ShapeDtypeStruct((xp.shape[0], wp.shape[1]),
                                       jnp.float32),
    )(xp, wp)
    return out[:n, :m]


def _logits_kern(a_ref, b_ref, o_ref):
    s = a_ref[...] + b_ref[...]
    o_ref[...] = jnp.where(s >= 0, s, _NEG * s)


def _edge_logits(la, lb):
    e = la.shape[0]
    ep = -(-e // 512) * 512
    a = jnp.pad(la, (0, ep - e)).reshape(-1, 512)
    b = jnp.pad(lb, (0, ep - e)).reshape(-1, 512)
    o = pl.pallas_call(
        _logits_kern,
        out_shape=jax.ShapeDtypeStruct(a.shape, jnp.float32))(a, b)
    return o.reshape(-1)[:e]


def _exp_kern(l_ref, m_ref, o_ref):
    o_ref[...] = jnp.exp(l_ref[...] - m_ref[...])


def _edge_exp(lg, mg):
    e = lg.shape[0]
    ep = -(-e // 512) * 512
    a = jnp.pad(lg, (0, ep - e)).reshape(-1, 512)
    b = jnp.pad(mg, (0, ep - e)).reshape(-1, 512)
    o = pl.pallas_call(
        _exp_kern,
        out_shape=jax.ShapeDtypeStruct(a.shape, jnp.float32))(a, b)
    return o.reshape(-1)[:e]


def _att_kern(e_ref, s_ref, o_ref):
    o_ref[...] = e_ref[...] / (s_ref[...] + 1e-12)


def _att_norm(e, s_g):
    n = e.shape[0]
    ep = -(-n // 512) * 512
    a = jnp.pad(e, (0, ep - n)).reshape(-1, 512)
    b = jnp.pad(s_g, (0, ep - n), constant_values=1.0).reshape(-1, 512)
    o = pl.pallas_call(
        _att_kern,
        out_shape=jax.ShapeDtypeStruct(a.shape, jnp.float32))(a, b)
    return o.reshape(-1)[:n]


def _comb1_kern(a_ref, o_ref):
    o_ref[...] = jnp.maximum(a_ref[...], 0.0)


def _comb2_kern(a_ref, b_ref, o_ref):
    o_ref[...] = jnp.maximum(a_ref[...], 0.0) + jnp.maximum(b_ref[...], 0.0)


def _comb3_kern(a_ref, b_ref, c_ref, o_ref):
    o_ref[...] = (jnp.maximum(a_ref[...], 0.0)
                  + jnp.maximum(b_ref[...], 0.0)
                  + jnp.maximum(c_ref[...], 0.0))


def _combine(*arrs):
    n = arrs[0].shape[0]
    tm = 512
    ps = [_pad_rows(a, tm) for a in arrs]
    g = ps[0].shape[0] // tm
    kern = {1: _comb1_kern, 2: _comb2_kern, 3: _comb3_kern}[len(arrs)]
    o = pl.pallas_call(
        kern,
        grid=(g,),
        in_specs=[pl.BlockSpec((tm, 128), lambda i: (i, 0))] * len(arrs),
        out_specs=pl.BlockSpec((tm, 128), lambda i: (i, 0)),
        out_shape=jax.ShapeDtypeStruct(ps[0].shape, jnp.float32),
    )(*ps)
    return o[:n]


def _att_reduce(la, lb, vals_g, seg, n, seg_sorted):
    lg = _edge_logits(la, lb)
    m = jax.ops.segment_max(lg, seg, num_segments=n,
                            indices_are_sorted=seg_sorted)
    m = jnp.where(jnp.isfinite(m), m, 0.0)
    e = _edge_exp(lg, jnp.take(m, seg))
    s = jax.ops.segment_sum(e, seg, num_segments=n,
                            indices_are_sorted=seg_sorted)
    att = _att_norm(e, jnp.take(s, seg))
    return jax.ops.segment_sum(att[:, None] * vals_g, seg, num_segments=n,
                               indices_are_sorted=seg_sorted)


def _split_a(a):
    d = a.shape[0] // 2
    return a[:d], a[d:]


def kernel(x_0, x_1, x_2, params, neighborhood_0_to_0, neighborhood_1_to_1,
           neighborhood_2_to_2, neighborhood_0_to_1, neighborhood_1_to_2):
    p = params
    n0, d = x_0.shape
    n1 = x_1.shape[0]
    n2 = x_2.shape[0]
    r00, c00 = neighborhood_0_to_0[0], neighborhood_0_to_0[1]
    r11, c11 = neighborhood_1_to_1[0], neighborhood_1_to_1[1]
    r22, c22 = neighborhood_2_to_2[0], neighborhood_2_to_2[1]
    r01, c01 = neighborhood_0_to_1[0], neighborhood_0_to_1[1]
    r12, c12 = neighborhood_1_to_2[0], neighborhood_1_to_2[1]

    a0t, a0b = _split_a(p["hbs0_l1_a"])
    a01t, a01b = _split_a(p["hbns01_l1_a"])
    a12t, a12b = _split_a(p["hbns12_l1_a"])
    w0 = p["hbs0_l1_w"]
    ws01, wt01 = p["hbns01_l1_ws"], p["hbns01_l1_wt"]
    ws12, wt12 = p["hbns12_l1_ws"], p["hbns12_l1_wt"]

    m0 = _matmul(x_0, jnp.concatenate(
        [w0, wt01, w0 @ a0t, w0 @ a0b, wt01 @ a01t, wt01 @ a01b], axis=1))
    msg00, t01 = m0[:, :d], m0[:, d:2 * d]
    u00, v00 = m0[:, 2 * d], m0[:, 2 * d + 1]
    t01u1, t01u2 = m0[:, 2 * d + 2], m0[:, 2 * d + 3]

    m1 = _matmul(x_1, jnp.concatenate(
        [ws01, wt12, ws01 @ a01t, ws01 @ a01b, wt12 @ a12t, wt12 @ a12b],
        axis=1))
    s01, t12 = m1[:, :d], m1[:, d:2 * d]
    s01u1, s01u2 = m1[:, 2 * d], m1[:, 2 * d + 1]
    t12u1, t12u2 = m1[:, 2 * d + 2], m1[:, 2 * d + 3]

    m2 = _matmul(x_2, jnp.concatenate(
        [ws12, ws12 @ a12t, ws12 @ a12b], axis=1))
    s12 = m2[:, :d]
    s12u1, s12u2 = m2[:, d], m2[:, d + 1]

    x0_l1 = _att_reduce(jnp.take(u00, r00), jnp.take(v00, c00),
                        jnp.take(msg00, c00, axis=0), r00, n0, True)
    x10_l1 = _att_reduce(jnp.take(s01u1, c01), jnp.take(t01u2, r01),
                         jnp.take(s01, c01, axis=0), r01, n0, True)
    x01_l1 = _att_reduce(jnp.take(t01u1, r01), jnp.take(s01u2, c01),
                         jnp.take(t01, r01, axis=0), c01, n1, False)
    x21_l1 = _att_reduce(jnp.take(s12u1, c12), jnp.take(t12u2, r12),
                         jnp.take(s12, c12, axis=0), r12, n1, True)
    x12_l1 = _att_reduce(jnp.take(t12u1, r12), jnp.take(s12u2, c12),
                         jnp.take(t12, r12, axis=0), c12, n2, False)

    h0 = _combine(x0_l1, x10_l1)
    h1 = _combine(x01_l1, x21_l1)
    h2 = _combine(x12_l1)

    b0t, b0b = _split_a(p["hbs0_l2_a"])
    b1t, b1b = _split_a(p["hbs1_l2_a"])
    b2t, b2b = _split_a(p["hbs2_l2_a"])
    b01t, b01b = _split_a(p["hbns01_l2_a"])
    b12t, b12b = _split_a(p["hbns12_l2_a"])
    q0 = p["hbs0_l2_w"]
    q1 = p["hbs1_l2_w"]
    q2 = p["hbs2_l2_w"]
    qs01, qt01 = p["hbns01_l2_ws"], p["hbns01_l2_wt"]
    qs12, qt12 = p["hbns12_l2_ws"], p["hbns12_l2_wt"]

    k0 = _matmul(h0, jnp.concatenate(
        [q0, qt01, q0 @ b0t, q0 @ b0b, qt01 @ b01t, qt01 @ b01b], axis=1))
    g00, gt01 = k0[:, :d], k0[:, d:2 * d]
    gu00, gv00 = k0[:, 2 * d], k0[:, 2 * d + 1]
    gt01u1, gt01u2 = k0[:, 2 * d + 2], k0[:, 2 * d + 3]

    k1 = _matmul(h1, jnp.concatenate(
        [qs01, q1, qt12, qs01 @ b01t, qs01 @ b01b, q1 @ b1t, q1 @ b1b,
         qt12 @ b12t, qt12 @ b12b], axis=1))
    gs01, g11, gt12 = k1[:, :d], k1[:, d:2 * d], k1[:, 2 * d:3 * d]
    gs01u1, gs01u2 = k1[:, 3 * d], k1[:, 3 * d + 1]
    gu11, gv11 = k1[:, 3 * d + 2], k1[:, 3 * d + 3]
    gt12u1, gt12u2 = k1[:, 3 * d + 4], k1[:, 3 * d + 5]

    k2 = _matmul(h2, jnp.concatenate(
        [qs12, q2, qs12 @ b12t, qs12 @ b12b, q2 @ b2t, q2 @ b2b], axis=1))
    gs12, g22 = k2[:, :d], k2[:, d:2 * d]
    gs12u1, gs12u2 = k2[:, 2 * d], k2[:, 2 * d + 1]
    gu22, gv22 = k2[:, 2 * d + 2], k2[:, 2 * d + 3]

    x0_l2 = _att_reduce(jnp.take(gu00, r00), jnp.take(gv00, c00),
                        jnp.take(g00, c00, axis=0), r00, n0, True)
    x10_l2 = _att_reduce(jnp.take(gs01u1, c01), jnp.take(gt01u2, r01),
                         jnp.take(gs01, c01, axis=0), r01, n0, True)
    x01_l2 = _att_reduce(jnp.take(gt01u1, r01), jnp.take(gs01u2, c01),
                         jnp.take(gt01, r01, axis=0), c01, n1, False)
    x1_l2 = _att_reduce(jnp.take(gu11, r11), jnp.take(gv11, c11),
                        jnp.take(g11, c11, axis=0), r11, n1, True)
    x21_l2 = _att_reduce(jnp.take(gs12u1, c12), jnp.take(gt12u2, r12),
                         jnp.take(gs12, c12, axis=0), r12, n1, True)
    x12_l2 = _att_reduce(jnp.take(gt12u1, r12), jnp.take(gs12u2, c12),
                         jnp.take(gt12, r12, axis=0), c12, n2, False)
    x2_l2 = _att_reduce(jnp.take(gu22, r22), jnp.take(gv22, c22),
                        jnp.take(g22, c22, axis=0), r22, n2, True)

    out_0 = _combine(x0_l2, x10_l2)
    out_1 = _combine(x01_l2, x1_l2, x21_l2)
    out_2 = _combine(x12_l2, x2_l2)
    return (out_0, out_1, out_2)
